# baseline (speedup 1.0000x reference)
import functools

import jax
import jax.numpy as jnp
from jax import lax
from jax.experimental import pallas as pl
from jax.experimental.pallas import tpu as pltpu
from jax.experimental.pallas import tpu_sc as plsc

VOCAB = 100000
EMBED = 16
NGRAM = 5
BATCH = 1024
FAN_IN = NGRAM * EMBED

NW = 32
N_IDX = BATCH * NGRAM
PER_W = N_IDX // NW
CHUNK = 80
NCH = PER_W // CHUNK

VB = 2048
GRID_V = (VOCAB + VB - 1) // VB


def _sc_gather(table, idx):
    mesh = plsc.VectorSubcoreMesh(core_axis_name="c", subcore_axis_name="s")

    @functools.partial(
        pl.kernel,
        out_type=jax.ShapeDtypeStruct((N_IDX, EMBED), jnp.float32),
        mesh=mesh,
        scratch_types=[
            pltpu.VMEM((NCH, CHUNK), jnp.int32),
            pltpu.VMEM((PER_W, EMBED), jnp.float32),
            pltpu.SemaphoreType.DMA,
        ],
        compiler_params=pltpu.CompilerParams(use_tc_tiling_on_sc=False),
    )
    def k(table_hbm, idx_hbm, out_hbm, idx_v, rows_v, sem):
        wid = lax.axis_index("s") * 2 + lax.axis_index("c")
        base = wid * PER_W
        for c in range(NCH):
            pltpu.sync_copy(idx_hbm.at[pl.ds(base + c * CHUNK, CHUNK)],
                            idx_v.at[c])
        copies = [
            pltpu.async_copy(table_hbm.at[idx_v.at[c]],
                             rows_v.at[pl.ds(c * CHUNK, CHUNK)], sem)
            for c in range(NCH)
        ]
        for cp in copies:
            cp.wait()
        pltpu.sync_copy(rows_v, out_hbm.at[pl.ds(base, PER_W)])

    return k(table, idx)


NBUF = 4
LAST = GRID_V - 1
TAIL = VOCAB - LAST * VB


def _tc_matmul(z1, W1, b2):
    def body(z_ref, w_ref, b_ref, o_hbm, bufs, tbuf, sems, tsem):
        i = pl.program_id(0)
        slot = lax.rem(i, NBUF)

        @pl.when(jnp.logical_and(i >= NBUF, i < LAST))
        def _():
            pltpu.make_async_copy(
                bufs.at[slot],
                o_hbm.at[:, pl.ds((i - NBUF) * VB, VB)],
                sems.at[slot]).wait()

        acc = lax.dot_general(
            z_ref[...], w_ref[...],
            (((1,), (1,)), ((), ())),
            preferred_element_type=jnp.float32,
        ) + b_ref[...]

        @pl.when(i < LAST)
        def _():
            bufs[slot] = acc
            pltpu.make_async_copy(
                bufs.at[slot],
                o_hbm.at[:, pl.ds(i * VB, VB)],
                sems.at[slot]).start()

        @pl.when(i == LAST)
        def _():
            tbuf[...] = acc[:, :TAIL]
            pltpu.make_async_copy(
                tbuf,
                o_hbm.at[:, pl.ds(LAST * VB, TAIL)],
                tsem).start()
            for k in range(NBUF):
                s = LAST - NBUF + k
                pltpu.make_async_copy(
                    bufs.at[s % NBUF],
                    o_hbm.at[:, pl.ds(s * VB, VB)],
                    sems.at[s % NBUF]).wait()
            pltpu.make_async_copy(
                tbuf,
                o_hbm.at[:, pl.ds(LAST * VB, TAIL)],
                tsem).wait()

    return pl.pallas_call(
        body,
        grid=(GRID_V,),
        in_specs=[
            pl.BlockSpec((BATCH, FAN_IN), lambda i: (0, 0)),
            pl.BlockSpec((VB, FAN_IN), lambda i: (i, 0)),
            pl.BlockSpec((1, VB), lambda i: (0, i)),
        ],
        out_specs=pl.BlockSpec(memory_space=pl.ANY),
        out_shape=jax.ShapeDtypeStruct((BATCH, VOCAB), jnp.float32),
        scratch_shapes=[
            pltpu.VMEM((NBUF, BATCH, VB), jnp.float32),
            pltpu.VMEM((BATCH, TAIL), jnp.float32),
            pltpu.SemaphoreType.DMA((NBUF,)),
            pltpu.SemaphoreType.DMA,
        ],
        compiler_params=pltpu.CompilerParams(
            dimension_semantics=("arbitrary",)),
    )(z1, W1, b2)


def kernel(inputs, emb_table, W1, b1):
    idx = inputs.reshape(-1).astype(jnp.int32)
    rows = _sc_gather(emb_table, idx)
    z1 = rows.reshape(BATCH, FAN_IN)
    return _tc_matmul(z1, W1, b1.reshape(1, VOCAB))


# speedup vs baseline: 1.0552x; 1.0552x over previous
import functools

import jax
import jax.numpy as jnp
from jax import lax
from jax.experimental import pallas as pl
from jax.experimental.pallas import tpu as pltpu
from jax.experimental.pallas import tpu_sc as plsc

VOCAB = 100000
EMBED = 16
NGRAM = 5
BATCH = 1024
FAN_IN = NGRAM * EMBED

NW = 32
N_IDX = BATCH * NGRAM
PER_W = N_IDX // NW
CHUNK = 80
NCH = PER_W // CHUNK

VB = 2048
GRID_V = (VOCAB + VB - 1) // VB


def _sc_gather(table, idx):
    mesh = plsc.VectorSubcoreMesh(core_axis_name="c", subcore_axis_name="s")

    @functools.partial(
        pl.kernel,
        out_type=jax.ShapeDtypeStruct((N_IDX, EMBED), jnp.float32),
        mesh=mesh,
        scratch_types=[
            pltpu.VMEM((NCH, CHUNK), jnp.int32),
            pltpu.VMEM((PER_W, EMBED), jnp.float32),
            pltpu.SemaphoreType.DMA,
        ],
        compiler_params=pltpu.CompilerParams(use_tc_tiling_on_sc=False),
    )
    def k(table_hbm, idx_hbm, out_hbm, idx_v, rows_v, sem):
        wid = lax.axis_index("s") * 2 + lax.axis_index("c")
        base = wid * PER_W
        for c in range(NCH):
            pltpu.sync_copy(idx_hbm.at[pl.ds(base + c * CHUNK, CHUNK)],
                            idx_v.at[c])
        copies = [
            pltpu.async_copy(table_hbm.at[idx_v.at[c]],
                             rows_v.at[pl.ds(c * CHUNK, CHUNK)], sem)
            for c in range(NCH)
        ]
        for cp in copies:
            cp.wait()
        pltpu.sync_copy(rows_v, out_hbm.at[pl.ds(base, PER_W)])

    return k(table, idx)


NBUF = 4
LAST = GRID_V - 1
TAIL = VOCAB - LAST * VB


def _tc_matmul(z1, W1, b2):
    def body(z_ref, w_ref, b_ref, o_hbm, bufs, tbuf, sems, tsem):
        i = pl.program_id(0)
        slot = lax.rem(i, NBUF)

        @pl.when(jnp.logical_and(i >= NBUF, i < LAST))
        def _():
            pltpu.make_async_copy(
                bufs.at[slot],
                o_hbm.at[:, pl.ds((i - NBUF) * VB, VB)],
                sems.at[slot]).wait()

        acc = lax.dot_general(
            z_ref[...], w_ref[...],
            (((1,), (1,)), ((), ())),
            preferred_element_type=jnp.float32,
        ) + b_ref[...]

        @pl.when(i < LAST)
        def _():
            bufs[slot] = acc
            pltpu.make_async_copy(
                bufs.at[slot],
                o_hbm.at[:, pl.ds(i * VB, VB)],
                sems.at[slot]).start()

        @pl.when(i == LAST)
        def _():
            tbuf[...] = acc[:, :TAIL]
            pltpu.make_async_copy(
                tbuf,
                o_hbm.at[:, pl.ds(LAST * VB, TAIL)],
                tsem).start()
            for k in range(NBUF):
                s = LAST - NBUF + k
                pltpu.make_async_copy(
                    bufs.at[s % NBUF],
                    o_hbm.at[:, pl.ds(s * VB, VB)],
                    sems.at[s % NBUF]).wait()
            pltpu.make_async_copy(
                tbuf,
                o_hbm.at[:, pl.ds(LAST * VB, TAIL)],
                tsem).wait()

    return pl.pallas_call(
        body,
        grid=(GRID_V,),
        in_specs=[
            pl.BlockSpec((BATCH, FAN_IN), lambda i: (0, 0)),
            pl.BlockSpec((VB, FAN_IN), lambda i: (i, 0)),
            pl.BlockSpec((1, VB), lambda i: (0, i)),
        ],
        out_specs=pl.BlockSpec(memory_space=pl.ANY),
        out_shape=jax.ShapeDtypeStruct((BATCH, VOCAB), jnp.float32),
        scratch_shapes=[
            pltpu.VMEM((NBUF, BATCH, VB), jnp.float32),
            pltpu.VMEM((BATCH, TAIL), jnp.float32),
            pltpu.SemaphoreType.DMA((NBUF,)),
            pltpu.SemaphoreType.DMA,
        ],
        compiler_params=pltpu.CompilerParams(
            dimension_semantics=("arbitrary",)),
    )(z1, W1, b2)


def kernel(inputs, emb_table, W1, b1):
    idx = inputs.reshape(-1).astype(jnp.int32)
    rows = jnp.take(emb_table, idx, axis=0)
    z1 = rows.reshape(BATCH, FAN_IN)
    return _tc_matmul(z1, W1, b1.reshape(1, VOCAB))


# speedup vs baseline: 1.0596x; 1.0042x over previous
import functools

import jax
import jax.numpy as jnp
from jax import lax
from jax.experimental import pallas as pl
from jax.experimental.pallas import tpu as pltpu
from jax.experimental.pallas import tpu_sc as plsc

VOCAB = 100000
EMBED = 16
NGRAM = 5
BATCH = 1024
FAN_IN = NGRAM * EMBED

NW = 32
N_IDX = BATCH * NGRAM
PER_W = N_IDX // NW
CHUNK = 80
NCH = PER_W // CHUNK

VB = 2048
GRID_V = (VOCAB + VB - 1) // VB


def _sc_gather(table, idx):
    mesh = plsc.VectorSubcoreMesh(core_axis_name="c", subcore_axis_name="s")

    @functools.partial(
        pl.kernel,
        out_type=jax.ShapeDtypeStruct((N_IDX, EMBED), jnp.float32),
        mesh=mesh,
        scratch_types=[
            pltpu.VMEM((NCH, CHUNK), jnp.int32),
            pltpu.VMEM((PER_W, EMBED), jnp.float32),
            pltpu.SemaphoreType.DMA,
        ],
        compiler_params=pltpu.CompilerParams(use_tc_tiling_on_sc=False),
    )
    def k(table_hbm, idx_hbm, out_hbm, idx_v, rows_v, sem):
        wid = lax.axis_index("s") * 2 + lax.axis_index("c")
        base = wid * PER_W
        for c in range(NCH):
            pltpu.sync_copy(idx_hbm.at[pl.ds(base + c * CHUNK, CHUNK)],
                            idx_v.at[c])
        copies = [
            pltpu.async_copy(table_hbm.at[idx_v.at[c]],
                             rows_v.at[pl.ds(c * CHUNK, CHUNK)], sem)
            for c in range(NCH)
        ]
        for cp in copies:
            cp.wait()
        pltpu.sync_copy(rows_v, out_hbm.at[pl.ds(base, PER_W)])

    return k(table, idx)


NBUF = 4
LAST = GRID_V - 1
TAIL = VOCAB - LAST * VB


def _tc_matmul(z1, W1, b2):
    def body(z_ref, w_ref, b_ref, o_hbm, bufs, tbuf, sems, tsem):
        i = pl.program_id(0)
        slot = lax.rem(i, NBUF)

        @pl.when(jnp.logical_and(i >= NBUF, i < LAST))
        def _():
            pltpu.make_async_copy(
                bufs.at[slot],
                o_hbm.at[:, pl.ds((i - NBUF) * VB, VB)],
                sems.at[slot]).wait()

        acc = lax.dot_general(
            z_ref[...].astype(jnp.bfloat16), w_ref[...].astype(jnp.bfloat16),
            (((1,), (1,)), ((), ())),
            preferred_element_type=jnp.float32,
        ) + b_ref[...]

        @pl.when(i < LAST)
        def _():
            bufs[slot] = acc
            pltpu.make_async_copy(
                bufs.at[slot],
                o_hbm.at[:, pl.ds(i * VB, VB)],
                sems.at[slot]).start()

        @pl.when(i == LAST)
        def _():
            tbuf[...] = acc[:, :TAIL]
            pltpu.make_async_copy(
                tbuf,
                o_hbm.at[:, pl.ds(LAST * VB, TAIL)],
                tsem).start()
            for k in range(NBUF):
                s = LAST - NBUF + k
                pltpu.make_async_copy(
                    bufs.at[s % NBUF],
                    o_hbm.at[:, pl.ds(s * VB, VB)],
                    sems.at[s % NBUF]).wait()
            pltpu.make_async_copy(
                tbuf,
                o_hbm.at[:, pl.ds(LAST * VB, TAIL)],
                tsem).wait()

    return pl.pallas_call(
        body,
        grid=(GRID_V,),
        in_specs=[
            pl.BlockSpec((BATCH, FAN_IN), lambda i: (0, 0)),
            pl.BlockSpec((VB, FAN_IN), lambda i: (i, 0)),
            pl.BlockSpec((1, VB), lambda i: (0, i)),
        ],
        out_specs=pl.BlockSpec(memory_space=pl.ANY),
        out_shape=jax.ShapeDtypeStruct((BATCH, VOCAB), jnp.float32),
        scratch_shapes=[
            pltpu.VMEM((NBUF, BATCH, VB), jnp.float32),
            pltpu.VMEM((BATCH, TAIL), jnp.float32),
            pltpu.SemaphoreType.DMA((NBUF,)),
            pltpu.SemaphoreType.DMA,
        ],
        compiler_params=pltpu.CompilerParams(
            dimension_semantics=("arbitrary",)),
    )(z1, W1, b2)


def kernel(inputs, emb_table, W1, b1):
    idx = inputs.reshape(-1).astype(jnp.int32)
    rows = jnp.take(emb_table, idx, axis=0)
    z1 = rows.reshape(BATCH, FAN_IN)
    return _tc_matmul(z1, W1, b1.reshape(1, VOCAB))


# speedup vs baseline: 1.1420x; 1.0778x over previous
import functools

import jax
import jax.numpy as jnp
from jax import lax
from jax.experimental import pallas as pl
from jax.experimental.pallas import tpu as pltpu
from jax.experimental.pallas import tpu_sc as plsc

VOCAB = 100000
EMBED = 16
NGRAM = 5
BATCH = 1024
FAN_IN = NGRAM * EMBED

NW = 32
N_IDX = BATCH * NGRAM
PER_W = N_IDX // NW
CHUNK = 80
NCH = PER_W // CHUNK

VB = 2048
GRID_V = (VOCAB + VB - 1) // VB


def _sc_gather(table, idx):
    mesh = plsc.VectorSubcoreMesh(core_axis_name="c", subcore_axis_name="s")

    @functools.partial(
        pl.kernel,
        out_type=jax.ShapeDtypeStruct((N_IDX, EMBED), jnp.float32),
        mesh=mesh,
        scratch_types=[
            pltpu.VMEM((NCH, CHUNK), jnp.int32),
            pltpu.VMEM((PER_W, EMBED), jnp.float32),
            pltpu.SemaphoreType.DMA,
        ],
        compiler_params=pltpu.CompilerParams(use_tc_tiling_on_sc=False),
    )
    def k(table_hbm, idx_hbm, out_hbm, idx_v, rows_v, sem):
        wid = lax.axis_index("s") * 2 + lax.axis_index("c")
        base = wid * PER_W
        for c in range(NCH):
            pltpu.sync_copy(idx_hbm.at[pl.ds(base + c * CHUNK, CHUNK)],
                            idx_v.at[c])
        copies = [
            pltpu.async_copy(table_hbm.at[idx_v.at[c]],
                             rows_v.at[pl.ds(c * CHUNK, CHUNK)], sem)
            for c in range(NCH)
        ]
        for cp in copies:
            cp.wait()
        pltpu.sync_copy(rows_v, out_hbm.at[pl.ds(base, PER_W)])

    return k(table, idx)


NBUF = 4
LAST = GRID_V - 1
TAIL = VOCAB - LAST * VB


def _tc_matmul(z1, W1, b2):
    def body(z_ref, w_ref, b_ref, o_hbm, bufs, tbuf, sems, tsem):
        i = pl.program_id(0)
        slot = lax.rem(i, NBUF)

        @pl.when(jnp.logical_and(i >= NBUF, i < LAST))
        def _():
            pltpu.make_async_copy(
                bufs.at[slot],
                o_hbm.at[:, pl.ds((i - NBUF) * VB, VB)],
                sems.at[slot]).wait()

        acc = lax.dot_general(
            z_ref[...], w_ref[...],
            (((1,), (0,)), ((), ())),
            preferred_element_type=jnp.float32,
        ) + b_ref[...]

        @pl.when(i < LAST)
        def _():
            bufs[slot] = acc
            pltpu.make_async_copy(
                bufs.at[slot],
                o_hbm.at[:, pl.ds(i * VB, VB)],
                sems.at[slot]).start()

        @pl.when(i == LAST)
        def _():
            tbuf[...] = acc[:, :TAIL]
            pltpu.make_async_copy(
                tbuf,
                o_hbm.at[:, pl.ds(LAST * VB, TAIL)],
                tsem).start()
            for k in range(NBUF):
                s = LAST - NBUF + k
                pltpu.make_async_copy(
                    bufs.at[s % NBUF],
                    o_hbm.at[:, pl.ds(s * VB, VB)],
                    sems.at[s % NBUF]).wait()
            pltpu.make_async_copy(
                tbuf,
                o_hbm.at[:, pl.ds(LAST * VB, TAIL)],
                tsem).wait()

    return pl.pallas_call(
        body,
        grid=(GRID_V,),
        in_specs=[
            pl.BlockSpec((BATCH, FAN_IN), lambda i: (0, 0)),
            pl.BlockSpec((FAN_IN, VB), lambda i: (0, i)),
            pl.BlockSpec((1, VB), lambda i: (0, i)),
        ],
        out_specs=pl.BlockSpec(memory_space=pl.ANY),
        out_shape=jax.ShapeDtypeStruct((BATCH, VOCAB), jnp.float32),
        scratch_shapes=[
            pltpu.VMEM((NBUF, BATCH, VB), jnp.float32),
            pltpu.VMEM((BATCH, TAIL), jnp.float32),
            pltpu.SemaphoreType.DMA((NBUF,)),
            pltpu.SemaphoreType.DMA,
        ],
        compiler_params=pltpu.CompilerParams(
            dimension_semantics=("arbitrary",)),
    )(z1, W1, b2)


def kernel(inputs, emb_table, W1, b1):
    idx = inputs.reshape(-1).astype(jnp.int32)
    rows = jnp.take(emb_table, idx, axis=0)
    z1 = rows.reshape(BATCH, FAN_IN)
    return _tc_matmul(z1, W1.T, b1.reshape(1, VOCAB))
